# Initial kernel scaffold; baseline (speedup 1.0000x reference)
import jax
import jax.numpy as jnp
from jax.experimental import pallas as pl


def kernel(x, w1, b1, w2, b2):
    raise NotImplementedError("write your pallas kernel here")


# speedup vs baseline: 1.1849x; 1.1849x over previous
import jax
import jax.numpy as jnp
from jax import lax
from jax.experimental import pallas as pl
from jax.experimental.pallas import tpu as pltpu


def _make_body(H, W, C, T):
    HW = H * W
    P = 128

    def body(x_ref, w1_ref, b1_ref, w2_ref, b2_ref, out_ref, pad_ref, col_ref):

        pad_ref[:, :P] = jnp.zeros((C, P), jnp.bfloat16)
        pad_ref[:, P + HW:] = jnp.zeros((C, P), jnp.bfloat16)

        colid = lax.broadcasted_iota(jnp.int32, (C, HW), 1) % W
        ok_left = colid >= 1
        ok_right = colid <= W - 2

        def conv3x3(a_bf, w2d, b):
            pad_ref[:, P:P + HW] = a_bf
            for k in range(9):
                oy, ox = k // 3 - 1, k % 3 - 1
                s = P + oy * W + ox
                src = pad_ref[:, s:s + HW]
                if ox == -1:
                    src = jnp.where(ok_left, src, jnp.bfloat16(0))
                elif ox == 1:
                    src = jnp.where(ok_right, src, jnp.bfloat16(0))
                col_ref[k * C:(k + 1) * C, :] = src
            return jnp.dot(w2d, col_ref[...],
                           preferred_element_type=jnp.float32) + b

        def block(t, r):
            y1 = jnp.maximum(conv3x3(r.astype(jnp.bfloat16),
                                     w1_ref[t], b1_ref[t]), 0.0)
            y2 = conv3x3(y1.astype(jnp.bfloat16), w2_ref[t], b2_ref[t])
            return r + y2

        out_ref[0] = lax.fori_loop(0, T, block, x_ref[0])

    return body


def kernel(x, w1, b1, w2, b2):
    N, C, H, W = x.shape
    T = w1.shape[0]
    HW = H * W
    P = 128

    w1m = jnp.transpose(w1, (0, 3, 1, 2)).reshape(T, C, 9 * C).astype(jnp.bfloat16)
    w2m = jnp.transpose(w2, (0, 3, 1, 2)).reshape(T, C, 9 * C).astype(jnp.bfloat16)
    b1m = jnp.transpose(b1, (0, 2, 1))
    b2m = jnp.transpose(b2, (0, 2, 1))

    xf = x.reshape(N, C, HW)
    out = pl.pallas_call(
        _make_body(H, W, C, T),
        out_shape=jax.ShapeDtypeStruct((N, C, HW), x.dtype),
        grid=(N,),
        in_specs=[
            pl.BlockSpec((1, C, HW), lambda n: (n, 0, 0)),
            pl.BlockSpec((T, C, 9 * C), lambda n: (0, 0, 0)),
            pl.BlockSpec((T, C, 1), lambda n: (0, 0, 0)),
            pl.BlockSpec((T, C, 9 * C), lambda n: (0, 0, 0)),
            pl.BlockSpec((T, C, 1), lambda n: (0, 0, 0)),
        ],
        out_specs=pl.BlockSpec((1, C, HW), lambda n: (n, 0, 0)),
        scratch_shapes=[
            pltpu.VMEM((C, HW + 2 * P), jnp.bfloat16),
            pltpu.VMEM((9 * C, HW), jnp.bfloat16),
        ],
        compiler_params=pltpu.CompilerParams(
            dimension_semantics=("parallel",)),
    )(xf, w1m, b1m, w2m, b2m)
    return out.reshape(N, C, H, W)
